# baseline (speedup 1.0000x reference)
import functools

import jax
import jax.numpy as jnp
from jax.experimental import pallas as pl
from jax.experimental.pallas import tpu as pltpu

K = 5
H_IN, W_IN = 32, 32
C1, H1, W1 = 6, 28, 28
HP1, WP1 = 14, 14
C2, H2, W2 = 24, 10, 10
HP2, WP2 = 5, 5
HID, OUT = 48, 10

RS1 = 96
RS2 = 128
BB = 2048


def _lenet_kernel(x_ref, band1_ref, b1_ref, band2_ref, b2_ref,
                  fc1_ref, bf1_ref, fc2_ref, bf2_ref, o_ref,
                  xb_ref, p1_ref, p2_ref):
    f32 = jnp.float32
    bf16 = jnp.bfloat16
    B = o_ref.shape[1]

    xb_ref[...] = x_ref[...].astype(bf16)

    band1 = band1_ref[...]
    z4 = jnp.zeros((RS1 - C1 * WP1, B), bf16)
    for i in range(HP1):
        e = jnp.dot(band1, xb_ref[pl.ds(64 * i, K * W_IN), :],
                    preferred_element_type=f32)
        o = jnp.dot(band1, xb_ref[pl.ds(64 * i + 32, K * W_IN), :],
                    preferred_element_type=f32)
        m = jnp.maximum(e, o)
        m = jnp.maximum(m[:C1 * WP1], m[C1 * WP1:])
        m = jnp.maximum(m + b1_ref[...], 0.0).astype(bf16)
        p1_ref[pl.ds(RS1 * i, RS1), :] = jnp.concatenate([m, z4], axis=0)

    band2 = band2_ref[...]
    z8 = jnp.zeros((RS2 - C2 * WP2, B), bf16)
    for i in range(HP2):
        e = jnp.dot(band2, p1_ref[pl.ds(2 * RS1 * i, K * RS1), :],
                    preferred_element_type=f32)
        o = jnp.dot(band2, p1_ref[pl.ds(2 * RS1 * i + RS1, K * RS1), :],
                    preferred_element_type=f32)
        m = jnp.maximum(e, o)
        m = jnp.maximum(m[:C2 * WP2], m[C2 * WP2:])
        m = jnp.maximum(m + b2_ref[...], 0.0).astype(bf16)
        p2_ref[pl.ds(RS2 * i, RS2), :] = jnp.concatenate([m, z8], axis=0)

    h = jnp.dot(fc1_ref[...], p2_ref[...], preferred_element_type=f32)
    h = jnp.maximum(h + bf1_ref[...], 0.0)
    out = jnp.dot(fc2_ref[...], h, preferred_element_type=f32)
    o_ref[...] = jnp.maximum(out + bf2_ref[...], 0.0)


def _band1_matrix(w1):
    ow = jnp.arange(W1)[:, None]
    iw = jnp.arange(W_IN)[None, :]
    d = iw - ow
    mask = (d >= 0) & (d < K)
    dc = jnp.clip(d, 0, K - 1)
    w = w1[:, 0, :, :]
    s = jnp.take(w, dc, axis=2) * mask[None, None]
    s = jnp.transpose(s, (0, 2, 1, 3))
    s = s.reshape(C1, W1, K * W_IN)
    even = s[:, 0::2].reshape(C1 * WP1, K * W_IN)
    odd = s[:, 1::2].reshape(C1 * WP1, K * W_IN)
    return jnp.concatenate([even, odd], axis=0)


def _band2_matrix(w2):
    ow = jnp.arange(W2)[:, None]
    iw = jnp.arange(WP1)[None, :]
    d = iw - ow
    mask = (d >= 0) & (d < K)
    dc = jnp.clip(d, 0, K - 1)
    s = jnp.take(w2, dc, axis=3) * mask[None, None, None]
    s = jnp.transpose(s, (0, 3, 2, 1, 4))
    s = s.reshape(C2, W2, K, C1 * WP1)
    s = jnp.pad(s, ((0, 0), (0, 0), (0, 0), (0, RS1 - C1 * WP1)))
    s = s.reshape(C2, W2, K * RS1)
    even = s[:, 0::2].reshape(C2 * WP2, K * RS1)
    odd = s[:, 1::2].reshape(C2 * WP2, K * RS1)
    return jnp.concatenate([even, odd], axis=0)


@functools.partial(jax.jit, static_argnums=(9,))
def _forward(x_t, band1, b1r, band2, b2r, fc1p, bf1, fc2, bf2, block_b):
    n_pad = x_t.shape[-1]
    grid = (n_pad // block_b,)
    return pl.pallas_call(
        _lenet_kernel,
        out_shape=jax.ShapeDtypeStruct((OUT, n_pad), jnp.float32),
        grid=grid,
        in_specs=[
            pl.BlockSpec((H_IN * W_IN, block_b), lambda n: (0, n)),
            pl.BlockSpec((C1 * WP1 * 2, K * W_IN), lambda n: (0, 0)),
            pl.BlockSpec((C1 * WP1, 1), lambda n: (0, 0)),
            pl.BlockSpec((C2 * WP2 * 2, K * RS1), lambda n: (0, 0)),
            pl.BlockSpec((C2 * WP2, 1), lambda n: (0, 0)),
            pl.BlockSpec((HID, HP2 * RS2), lambda n: (0, 0)),
            pl.BlockSpec((HID, 1), lambda n: (0, 0)),
            pl.BlockSpec((OUT, HID), lambda n: (0, 0)),
            pl.BlockSpec((OUT, 1), lambda n: (0, 0)),
        ],
        out_specs=pl.BlockSpec((OUT, block_b), lambda n: (0, n)),
        scratch_shapes=[
            pltpu.VMEM((H_IN * W_IN, block_b), jnp.bfloat16),
            pltpu.VMEM((HP1 * RS1, block_b), jnp.bfloat16),
            pltpu.VMEM((HP2 * RS2, block_b), jnp.bfloat16),
        ],
        compiler_params=pltpu.CompilerParams(
            dimension_semantics=("parallel",)),
    )(x_t, band1, b1r, band2, b2r, fc1p, bf1, fc2, bf2)


def kernel(x_nchw, conv1_w, conv1_b, conv2_w, conv2_b,
           fc1_w, fc1_b, fc2_w, fc2_b):
    N = x_nchw.shape[0]
    bb = BB if N >= BB else N
    n_pad = -(-N // bb) * bb

    x = x_nchw.reshape(N, H_IN * W_IN)
    if n_pad != N:
        x = jnp.pad(x, ((0, n_pad - N), (0, 0)))
    x = x.T

    band1 = _band1_matrix(conv1_w).astype(jnp.bfloat16)
    band2 = _band2_matrix(conv2_w).astype(jnp.bfloat16)
    b1r = jnp.repeat(conv1_b, WP1).reshape(C1 * WP1, 1)
    b2r = jnp.repeat(conv2_b, WP2).reshape(C2 * WP2, 1)
    fc1p = fc1_w.reshape(HID, C2, HP2, WP2).transpose(0, 2, 1, 3)
    fc1p = fc1p.reshape(HID, HP2, C2 * WP2)
    fc1p = jnp.pad(fc1p, ((0, 0), (0, 0), (0, RS2 - C2 * WP2)))
    fc1p = fc1p.reshape(HID, HP2 * RS2).astype(jnp.bfloat16)
    bf1 = fc1_b.reshape(HID, 1)
    bf2 = fc2_b.reshape(OUT, 1)

    out = _forward(x, band1, b1r, band2, b2r, fc1p, bf1, fc2_w, bf2, bb)
    return out[:, :N].T


# speedup vs baseline: 1.0078x; 1.0078x over previous
import functools

import jax
import jax.numpy as jnp
from jax.experimental import pallas as pl
from jax.experimental.pallas import tpu as pltpu

K = 5
H_IN, W_IN = 32, 32
C1, H1, W1 = 6, 28, 28
HP1, WP1 = 14, 14
C2, H2, W2 = 24, 10, 10
HP2, WP2 = 5, 5
HID, OUT = 48, 10

RS1 = 96
RS2 = 128
BB = 1024


def _lenet_kernel(x_ref, band1_ref, b1_ref, band2_ref, b2_ref,
                  fc1_ref, bf1_ref, fc2_ref, bf2_ref, o_ref,
                  xb_ref, p1_ref, p2_ref):
    f32 = jnp.float32
    bf16 = jnp.bfloat16
    B = o_ref.shape[1]

    xb_ref[...] = x_ref[...].astype(bf16)

    band1 = band1_ref[...]
    z4 = jnp.zeros((RS1 - C1 * WP1, B), bf16)
    for i in range(HP1):
        e = jnp.dot(band1, xb_ref[pl.ds(64 * i, K * W_IN), :],
                    preferred_element_type=f32)
        o = jnp.dot(band1, xb_ref[pl.ds(64 * i + 32, K * W_IN), :],
                    preferred_element_type=f32)
        m = jnp.maximum(e, o)
        m = jnp.maximum(m[:C1 * WP1], m[C1 * WP1:])
        m = jnp.maximum(m + b1_ref[...], 0.0).astype(bf16)
        p1_ref[pl.ds(RS1 * i, RS1), :] = jnp.concatenate([m, z4], axis=0)

    band2 = band2_ref[...]
    z8 = jnp.zeros((RS2 - C2 * WP2, B), bf16)
    for i in range(HP2):
        e = jnp.dot(band2, p1_ref[pl.ds(2 * RS1 * i, K * RS1), :],
                    preferred_element_type=f32)
        o = jnp.dot(band2, p1_ref[pl.ds(2 * RS1 * i + RS1, K * RS1), :],
                    preferred_element_type=f32)
        m = jnp.maximum(e, o)
        m = jnp.maximum(m[:C2 * WP2], m[C2 * WP2:])
        m = jnp.maximum(m + b2_ref[...], 0.0).astype(bf16)
        p2_ref[pl.ds(RS2 * i, RS2), :] = jnp.concatenate([m, z8], axis=0)

    h = jnp.dot(fc1_ref[...], p2_ref[...], preferred_element_type=f32)
    h = jnp.maximum(h + bf1_ref[...], 0.0)
    out = jnp.dot(fc2_ref[...], h, preferred_element_type=f32)
    o_ref[...] = jnp.maximum(out + bf2_ref[...], 0.0)


def _band1_matrix(w1):
    ow = jnp.arange(W1)[:, None]
    iw = jnp.arange(W_IN)[None, :]
    d = iw - ow
    mask = (d >= 0) & (d < K)
    dc = jnp.clip(d, 0, K - 1)
    w = w1[:, 0, :, :]
    s = jnp.take(w, dc, axis=2) * mask[None, None]
    s = jnp.transpose(s, (0, 2, 1, 3))
    s = s.reshape(C1, W1, K * W_IN)
    even = s[:, 0::2].reshape(C1 * WP1, K * W_IN)
    odd = s[:, 1::2].reshape(C1 * WP1, K * W_IN)
    return jnp.concatenate([even, odd], axis=0)


def _band2_matrix(w2):
    ow = jnp.arange(W2)[:, None]
    iw = jnp.arange(WP1)[None, :]
    d = iw - ow
    mask = (d >= 0) & (d < K)
    dc = jnp.clip(d, 0, K - 1)
    s = jnp.take(w2, dc, axis=3) * mask[None, None, None]
    s = jnp.transpose(s, (0, 3, 2, 1, 4))
    s = s.reshape(C2, W2, K, C1 * WP1)
    s = jnp.pad(s, ((0, 0), (0, 0), (0, 0), (0, RS1 - C1 * WP1)))
    s = s.reshape(C2, W2, K * RS1)
    even = s[:, 0::2].reshape(C2 * WP2, K * RS1)
    odd = s[:, 1::2].reshape(C2 * WP2, K * RS1)
    return jnp.concatenate([even, odd], axis=0)


@functools.partial(jax.jit, static_argnums=(9,))
def _forward(x_t, band1, b1r, band2, b2r, fc1p, bf1, fc2, bf2, block_b):
    n_pad = x_t.shape[-1]
    grid = (n_pad // block_b,)
    return pl.pallas_call(
        _lenet_kernel,
        out_shape=jax.ShapeDtypeStruct((OUT, n_pad), jnp.float32),
        grid=grid,
        in_specs=[
            pl.BlockSpec((H_IN * W_IN, block_b), lambda n: (0, n)),
            pl.BlockSpec((C1 * WP1 * 2, K * W_IN), lambda n: (0, 0)),
            pl.BlockSpec((C1 * WP1, 1), lambda n: (0, 0)),
            pl.BlockSpec((C2 * WP2 * 2, K * RS1), lambda n: (0, 0)),
            pl.BlockSpec((C2 * WP2, 1), lambda n: (0, 0)),
            pl.BlockSpec((HID, HP2 * RS2), lambda n: (0, 0)),
            pl.BlockSpec((HID, 1), lambda n: (0, 0)),
            pl.BlockSpec((OUT, HID), lambda n: (0, 0)),
            pl.BlockSpec((OUT, 1), lambda n: (0, 0)),
        ],
        out_specs=pl.BlockSpec((OUT, block_b), lambda n: (0, n)),
        scratch_shapes=[
            pltpu.VMEM((H_IN * W_IN, block_b), jnp.bfloat16),
            pltpu.VMEM((HP1 * RS1, block_b), jnp.bfloat16),
            pltpu.VMEM((HP2 * RS2, block_b), jnp.bfloat16),
        ],
        compiler_params=pltpu.CompilerParams(
            dimension_semantics=("parallel",)),
    )(x_t, band1, b1r, band2, b2r, fc1p, bf1, fc2, bf2)


def kernel(x_nchw, conv1_w, conv1_b, conv2_w, conv2_b,
           fc1_w, fc1_b, fc2_w, fc2_b):
    N = x_nchw.shape[0]
    bb = BB if N >= BB else N
    n_pad = -(-N // bb) * bb

    x = x_nchw.reshape(N, H_IN * W_IN)
    if n_pad != N:
        x = jnp.pad(x, ((0, n_pad - N), (0, 0)))
    x = x.T

    band1 = _band1_matrix(conv1_w).astype(jnp.bfloat16)
    band2 = _band2_matrix(conv2_w).astype(jnp.bfloat16)
    b1r = jnp.repeat(conv1_b, WP1).reshape(C1 * WP1, 1)
    b2r = jnp.repeat(conv2_b, WP2).reshape(C2 * WP2, 1)
    fc1p = fc1_w.reshape(HID, C2, HP2, WP2).transpose(0, 2, 1, 3)
    fc1p = fc1p.reshape(HID, HP2, C2 * WP2)
    fc1p = jnp.pad(fc1p, ((0, 0), (0, 0), (0, RS2 - C2 * WP2)))
    fc1p = fc1p.reshape(HID, HP2 * RS2).astype(jnp.bfloat16)
    bf1 = fc1_b.reshape(HID, 1)
    bf2 = fc2_b.reshape(OUT, 1)

    out = _forward(x, band1, b1r, band2, b2r, fc1p, bf1, fc2_w, bf2, bb)
    return out[:, :N].T
